# baseline (speedup 1.0000x reference)
import functools
import math

import jax
import jax.numpy as jnp
from jax.experimental import pallas as pl
from jax.experimental.pallas import tpu as pltpu

_INTERPRET = False

_TN = 3584


def _gelu(x):
    return 0.5 * x * (1.0 + jax.lax.erf(x * (1.0 / math.sqrt(2.0))))


def _dotc(w, x):
    return jax.lax.dot_general(
        w, x, (((0,), (0,)), ((), ())), preferred_element_type=jnp.float32
    )


def _k1_body(x_ref, few_ref, feb_ref, g_ref):
    j = pl.program_id(1)
    ft = _gelu(_dotc(few_ref[...], x_ref[0]) + feb_ref[0][:, None])
    s = jnp.sum(ft, axis=1)

    @pl.when(j == 0)
    def _():
        g_ref[0, 0, :] = s

    @pl.when(j > 0)
    def _():
        g_ref[0, 0, :] = g_ref[0, 0, :] + s


def _kg_body(inv_p, g_ref, w1_ref, b1_ref, cw1_ref, cb1_ref, cw2_ref, cb2_ref,
             w2_ref, b2_ref, w3_ref, b3_ref, out_ref):
    g = g_ref[:, 0, :] * inv_p
    z = _gelu(jnp.dot(g, w1_ref[...], preferred_element_type=jnp.float32)
              + b1_ref[...])
    a = _gelu(jnp.dot(z, cw1_ref[...], preferred_element_type=jnp.float32)
              + cb1_ref[...])
    a = jnp.dot(a, cw2_ref[...], preferred_element_type=jnp.float32) + cb2_ref[...]
    z = z * jax.nn.sigmoid(2.0 * a)
    z = _gelu(jnp.dot(z, w2_ref[...], preferred_element_type=jnp.float32)
              + b2_ref[...])
    s = jnp.dot(z, w3_ref[...], preferred_element_type=jnp.float32) + b3_ref[...]
    idx = jax.lax.broadcasted_iota(jnp.int32, s.shape, 1)
    m1 = jnp.max(s, axis=1, keepdims=True)
    i1 = jnp.min(jnp.where(s >= m1, idx, 127), axis=1, keepdims=True)
    s2 = jnp.where(idx == i1, -jnp.inf, s)
    m2 = jnp.max(s2, axis=1, keepdims=True)
    i2 = jnp.min(jnp.where(s2 >= m2, idx, 127), axis=1, keepdims=True)
    d = jnp.exp((m2 - m1) * 0.5)
    w1 = 1.0 / (1.0 + d)
    w2 = d / (1.0 + d)
    out = jnp.where(idx == 0, w1,
          jnp.where(idx == 1, w2,
          jnp.where(idx == 2, i1.astype(jnp.float32),
          jnp.where(idx == 3, i2.astype(jnp.float32), 0.0))))
    out_ref[...] = out


def _k2_body(ti_ref, tw_ref, x_ref, few_ref, feb_ref, sw_ref, sb_ref,
             ewa_ref, ewb_ref, eba_ref, ebb_ref, out_ref):
    b = pl.program_id(0)
    bf16 = jnp.bfloat16
    xb = x_ref[0].astype(bf16)
    ft = _gelu(_dotc(few_ref[...].astype(bf16), xb) + feb_ref[0][:, None])
    ftb = ft.astype(bf16)
    acc = 0.5 * _gelu(_dotc(sw_ref[0].astype(bf16), ftb) + sb_ref[0, 0][:, None])
    acc = acc + 0.5 * _gelu(_dotc(sw_ref[1].astype(bf16), ftb) + sb_ref[1, 0][:, None])
    w0 = tw_ref[b, 0]
    w1 = tw_ref[b, 1]
    acc = acc + w0 * _gelu(_dotc(ewa_ref[0].astype(bf16), ftb) + eba_ref[0, 0][:, None])
    acc = acc + w1 * _gelu(_dotc(ewb_ref[0].astype(bf16), ftb) + ebb_ref[0, 0][:, None])
    out_ref[0] = acc


def kernel(x, fe_w, fe_b, g_w1, g_b1, bn1_g, bn1_b, ca_w1, ca_b1, ca_w2, ca_b2,
           g_w2, g_b2, bn2_g, bn2_b, g_w3, g_b3, shared_w, shared_b,
           expert_w, expert_b):
    B, C, H, W = x.shape
    P = H * W
    hidden = fe_w.shape[1]
    h2 = g_w1.shape[1]
    E = expert_w.shape[0]
    TN = _TN
    nP = P // TN
    f32 = jnp.float32

    x3 = x.reshape(B, C, P)
    feb2 = fe_b.reshape(1, hidden)

    g_sum = pl.pallas_call(
        _k1_body,
        grid=(B, nP),
        in_specs=[
            pl.BlockSpec((1, C, TN), lambda b, j: (b, 0, j)),
            pl.BlockSpec((C, hidden), lambda b, j: (0, 0)),
            pl.BlockSpec((1, hidden), lambda b, j: (0, 0)),
        ],
        out_specs=pl.BlockSpec((1, 1, hidden), lambda b, j: (b, 0, 0)),
        out_shape=jax.ShapeDtypeStruct((B, 1, hidden), f32),
        interpret=_INTERPRET,
    )(x3, fe_w, feb2)

    c = 1.0 / math.sqrt(1.0 + 1e-5)
    s1 = bn1_g * c
    w1f = g_w1 * s1[None, :]
    b1f = (g_b1 * s1 + bn1_b).reshape(1, h2)
    s2 = bn2_g * c
    w2f = g_w2 * s2[None, :]
    b2f = (g_b2 * s2 + bn2_b).reshape(1, hidden)

    red = ca_w1.shape[1]
    cw1p = jnp.pad(ca_w1, ((0, 0), (0, 128 - red)))
    cb1p = jnp.pad(ca_b1, (0, 128 - red)).reshape(1, 128)
    cw2p = jnp.pad(ca_w2, ((0, 128 - red), (0, 0)))
    cb2r = ca_b2.reshape(1, h2)
    w3p = jnp.pad(g_w3, ((0, 0), (0, 128 - E)))
    b3p = jnp.pad(g_b3, (0, 128 - E), constant_values=-1e9).reshape(1, 128)

    gout = pl.pallas_call(
        functools.partial(_kg_body, 1.0 / P),
        out_shape=jax.ShapeDtypeStruct((B, 128), f32),
        interpret=_INTERPRET,
    )(g_sum, w1f, b1f, cw1p, cb1p, cw2p, cb2r, w2f, b2f, w3p, b3p)

    top_w = gout[:, 0:2]
    top_i = gout[:, 2:4].astype(jnp.int32)

    eb3 = expert_b.reshape(E, 1, hidden)
    sb3 = shared_b.reshape(2, 1, hidden)

    out = pl.pallas_call(
        _k2_body,
        grid_spec=pltpu.PrefetchScalarGridSpec(
            num_scalar_prefetch=2,
            grid=(B, nP),
            in_specs=[
                pl.BlockSpec((1, C, TN), lambda b, j, ti, tw: (b, 0, j)),
                pl.BlockSpec((C, hidden), lambda b, j, ti, tw: (0, 0)),
                pl.BlockSpec((1, hidden), lambda b, j, ti, tw: (0, 0)),
                pl.BlockSpec((2, hidden, hidden), lambda b, j, ti, tw: (0, 0, 0)),
                pl.BlockSpec((2, 1, hidden), lambda b, j, ti, tw: (0, 0, 0)),
                pl.BlockSpec((1, hidden, hidden),
                             lambda b, j, ti, tw: (ti[b, 0], 0, 0)),
                pl.BlockSpec((1, hidden, hidden),
                             lambda b, j, ti, tw: (ti[b, 1], 0, 0)),
                pl.BlockSpec((1, 1, hidden), lambda b, j, ti, tw: (ti[b, 0], 0, 0)),
                pl.BlockSpec((1, 1, hidden), lambda b, j, ti, tw: (ti[b, 1], 0, 0)),
            ],
            out_specs=pl.BlockSpec((1, hidden, TN), lambda b, j, ti, tw: (b, 0, j)),
        ),
        out_shape=jax.ShapeDtypeStruct((B, hidden, P), f32),
        interpret=_INTERPRET,
    )(top_i, top_w, x3, fe_w, feb2, shared_w, sb3, expert_w, expert_w,
      expert_b.reshape(E, 1, hidden), eb3)

    return out.reshape(B, hidden, H, W)


# speedup vs baseline: 1.1076x; 1.1076x over previous
import functools
import math

import jax
import jax.numpy as jnp
from jax.experimental import pallas as pl
from jax.experimental.pallas import tpu as pltpu

_INTERPRET = False

_TN = 7168


def _gelu(x):
    return 0.5 * x * (1.0 + jax.lax.erf(x * (1.0 / math.sqrt(2.0))))


def _dotc(w, x):
    return jax.lax.dot_general(
        w, x, (((0,), (0,)), ((), ())), preferred_element_type=jnp.float32
    )


def _k1_body(x_ref, few_ref, feb_ref, g_ref, ft_ref):
    j = pl.program_id(1)
    ft = _gelu(_dotc(few_ref[...], x_ref[0]) + feb_ref[0][:, None])
    ft_ref[0] = ft.astype(jnp.bfloat16)
    s = jnp.sum(ft, axis=1)

    @pl.when(j == 0)
    def _():
        g_ref[0, 0, :] = s

    @pl.when(j > 0)
    def _():
        g_ref[0, 0, :] = g_ref[0, 0, :] + s


def _kg_body(inv_p, g_ref, w1_ref, b1_ref, cw1_ref, cb1_ref, cw2_ref, cb2_ref,
             w2_ref, b2_ref, w3_ref, b3_ref, out_ref):
    g = g_ref[:, 0, :] * inv_p
    z = _gelu(jnp.dot(g, w1_ref[...], preferred_element_type=jnp.float32)
              + b1_ref[...])
    a = _gelu(jnp.dot(z, cw1_ref[...], preferred_element_type=jnp.float32)
              + cb1_ref[...])
    a = jnp.dot(a, cw2_ref[...], preferred_element_type=jnp.float32) + cb2_ref[...]
    z = z * jax.nn.sigmoid(2.0 * a)
    z = _gelu(jnp.dot(z, w2_ref[...], preferred_element_type=jnp.float32)
              + b2_ref[...])
    s = jnp.dot(z, w3_ref[...], preferred_element_type=jnp.float32) + b3_ref[...]
    idx = jax.lax.broadcasted_iota(jnp.int32, s.shape, 1)
    m1 = jnp.max(s, axis=1, keepdims=True)
    i1 = jnp.min(jnp.where(s >= m1, idx, 127), axis=1, keepdims=True)
    s2 = jnp.where(idx == i1, -jnp.inf, s)
    m2 = jnp.max(s2, axis=1, keepdims=True)
    i2 = jnp.min(jnp.where(s2 >= m2, idx, 127), axis=1, keepdims=True)
    d = jnp.exp((m2 - m1) * 0.5)
    w1 = 1.0 / (1.0 + d)
    w2 = d / (1.0 + d)
    out = jnp.where(idx == 0, w1,
          jnp.where(idx == 1, w2,
          jnp.where(idx == 2, i1.astype(jnp.float32),
          jnp.where(idx == 3, i2.astype(jnp.float32), 0.0))))
    out_ref[...] = out


def _k2_body(ti_ref, tw_ref, ft_ref, sw_ref, sb_ref,
             ewa_ref, ewb_ref, eba_ref, ebb_ref, out_ref):
    b = pl.program_id(0)
    ftb = ft_ref[0]
    acc = 0.5 * _gelu(_dotc(sw_ref[0], ftb) + sb_ref[0, 0][:, None])
    acc = acc + 0.5 * _gelu(_dotc(sw_ref[1], ftb) + sb_ref[1, 0][:, None])
    w0 = tw_ref[b, 0]
    w1 = tw_ref[b, 1]
    acc = acc + w0 * _gelu(_dotc(ewa_ref[0], ftb) + eba_ref[0, 0][:, None])
    acc = acc + w1 * _gelu(_dotc(ewb_ref[0], ftb) + ebb_ref[0, 0][:, None])
    out_ref[0] = acc


def kernel(x, fe_w, fe_b, g_w1, g_b1, bn1_g, bn1_b, ca_w1, ca_b1, ca_w2, ca_b2,
           g_w2, g_b2, bn2_g, bn2_b, g_w3, g_b3, shared_w, shared_b,
           expert_w, expert_b):
    B, C, H, W = x.shape
    P = H * W
    hidden = fe_w.shape[1]
    h2 = g_w1.shape[1]
    E = expert_w.shape[0]
    TN = _TN
    nP = P // TN
    f32 = jnp.float32
    bf16 = jnp.bfloat16

    x3 = x.reshape(B, C, P)
    feb2 = fe_b.reshape(1, hidden)

    g_sum, ft_hbm = pl.pallas_call(
        _k1_body,
        grid=(B, nP),
        in_specs=[
            pl.BlockSpec((1, C, TN), lambda b, j: (b, 0, j)),
            pl.BlockSpec((C, hidden), lambda b, j: (0, 0)),
            pl.BlockSpec((1, hidden), lambda b, j: (0, 0)),
        ],
        out_specs=[
            pl.BlockSpec((1, 1, hidden), lambda b, j: (b, 0, 0)),
            pl.BlockSpec((1, hidden, TN), lambda b, j: (b, 0, j)),
        ],
        out_shape=[
            jax.ShapeDtypeStruct((B, 1, hidden), f32),
            jax.ShapeDtypeStruct((B, hidden, P), bf16),
        ],
        interpret=_INTERPRET,
    )(x3, fe_w, feb2)

    c = 1.0 / math.sqrt(1.0 + 1e-5)
    s1 = bn1_g * c
    w1f = g_w1 * s1[None, :]
    b1f = (g_b1 * s1 + bn1_b).reshape(1, h2)
    s2 = bn2_g * c
    w2f = g_w2 * s2[None, :]
    b2f = (g_b2 * s2 + bn2_b).reshape(1, hidden)

    red = ca_w1.shape[1]
    cw1p = jnp.pad(ca_w1, ((0, 0), (0, 128 - red)))
    cb1p = jnp.pad(ca_b1, (0, 128 - red)).reshape(1, 128)
    cw2p = jnp.pad(ca_w2, ((0, 128 - red), (0, 0)))
    cb2r = ca_b2.reshape(1, h2)
    w3p = jnp.pad(g_w3, ((0, 0), (0, 128 - E)))
    b3p = jnp.pad(g_b3, (0, 128 - E), constant_values=-1e9).reshape(1, 128)

    gout = pl.pallas_call(
        functools.partial(_kg_body, 1.0 / P),
        out_shape=jax.ShapeDtypeStruct((B, 128), f32),
        interpret=_INTERPRET,
    )(g_sum, w1f, b1f, cw1p, cb1p, cw2p, cb2r, w2f, b2f, w3p, b3p)

    top_w = gout[:, 0:2]
    top_i = gout[:, 2:4].astype(jnp.int32)

    swb = shared_w.astype(bf16)
    ewb = expert_w.astype(bf16)
    eb3 = expert_b.reshape(E, 1, hidden)
    sb3 = shared_b.reshape(2, 1, hidden)

    out = pl.pallas_call(
        _k2_body,
        grid_spec=pltpu.PrefetchScalarGridSpec(
            num_scalar_prefetch=2,
            grid=(B, nP),
            in_specs=[
                pl.BlockSpec((1, hidden, TN), lambda b, j, ti, tw: (b, 0, j)),
                pl.BlockSpec((2, hidden, hidden), lambda b, j, ti, tw: (0, 0, 0)),
                pl.BlockSpec((2, 1, hidden), lambda b, j, ti, tw: (0, 0, 0)),
                pl.BlockSpec((1, hidden, hidden),
                             lambda b, j, ti, tw: (ti[b, 0], 0, 0)),
                pl.BlockSpec((1, hidden, hidden),
                             lambda b, j, ti, tw: (ti[b, 1], 0, 0)),
                pl.BlockSpec((1, 1, hidden), lambda b, j, ti, tw: (ti[b, 0], 0, 0)),
                pl.BlockSpec((1, 1, hidden), lambda b, j, ti, tw: (ti[b, 1], 0, 0)),
            ],
            out_specs=pl.BlockSpec((1, hidden, TN), lambda b, j, ti, tw: (b, 0, j)),
        ),
        out_shape=jax.ShapeDtypeStruct((B, hidden, P), f32),
        interpret=_INTERPRET,
    )(top_i, top_w, ft_hbm, swb, sb3, ewb, ewb, eb3, eb3)

    return out.reshape(B, hidden, H, W)


# speedup vs baseline: 1.1671x; 1.0537x over previous
import functools
import math

import jax
import jax.numpy as jnp
from jax.experimental import pallas as pl
from jax.experimental.pallas import tpu as pltpu

_INTERPRET = False

_NP = 7


def _gelu(x):
    return 0.5 * x * (1.0 + jax.lax.erf(x * (1.0 / math.sqrt(2.0))))


def _dotc(w, x):
    return jax.lax.dot_general(
        w, x, (((0,), (0,)), ((), ())), preferred_element_type=jnp.float32
    )


def _dot(a, b):
    return jnp.dot(a, b, preferred_element_type=jnp.float32)


def _fused_body(x_ref, few_ref, feb_ref,
                w1_ref, b1_ref, cw1_ref, cb1_ref, cw2_ref, cb2_ref,
                w2_ref, b2_ref, w3_ref, b3_ref,
                sw_ref, sb_ref, ew_ref, eb_ref,
                out_ref,
                ft_ref, acc_ref, ti_ref, tw_ref,
                *, nB, TN, inv_p):
    b = pl.program_id(0)
    j = pl.program_id(1)

    @pl.when((b > 0) & (j == 0))
    def _():
        g = acc_ref[...] * inv_p
        z = _gelu(_dot(g, w1_ref[...]) + b1_ref[...])
        a = _gelu(_dot(z, cw1_ref[...]) + cb1_ref[...])
        a = _dot(a, cw2_ref[...]) + cb2_ref[...]
        z = z * jax.nn.sigmoid(2.0 * a)
        z = _gelu(_dot(z, w2_ref[...]) + b2_ref[...])
        s = _dot(z, w3_ref[...]) + b3_ref[...]
        idx = jax.lax.broadcasted_iota(jnp.int32, s.shape, 1)
        m1 = jnp.max(s)
        i1 = jnp.min(jnp.where(s >= m1, idx, 127))
        s2 = jnp.where(idx == i1, -jnp.inf, s)
        m2 = jnp.max(s2)
        i2 = jnp.min(jnp.where(s2 >= m2, idx, 127))
        d = jnp.exp((m2 - m1) * 0.5)
        ti_ref[0] = i1
        ti_ref[1] = i2
        tw_ref[0] = 1.0 / (1.0 + d)
        tw_ref[1] = d / (1.0 + d)

    @pl.when(b < nB)
    def _():
        ft = _gelu(_dotc(few_ref[...], x_ref[0]) + feb_ref[0][:, None])
        ft_ref[b % 2, :, pl.ds(j * TN, TN)] = ft.astype(jnp.bfloat16)
        s = jnp.sum(ft, axis=1)[None, :]

        @pl.when(j == 0)
        def _():
            acc_ref[...] = s

        @pl.when(j > 0)
        def _():
            acc_ref[...] = acc_ref[...] + s

    @pl.when(b > 0)
    def _():
        ftb = ft_ref[(b + 1) % 2, :, pl.ds(j * TN, TN)]
        acc = 0.5 * _gelu(_dotc(sw_ref[0], ftb) + sb_ref[0, 0][:, None])
        acc = acc + 0.5 * _gelu(_dotc(sw_ref[1], ftb) + sb_ref[1, 0][:, None])
        i0 = ti_ref[0]
        i1 = ti_ref[1]
        acc = acc + tw_ref[0] * _gelu(_dotc(ew_ref[i0], ftb)
                                      + eb_ref[i0, 0][:, None])
        acc = acc + tw_ref[1] * _gelu(_dotc(ew_ref[i1], ftb)
                                      + eb_ref[i1, 0][:, None])
        out_ref[0] = acc


def kernel(x, fe_w, fe_b, g_w1, g_b1, bn1_g, bn1_b, ca_w1, ca_b1, ca_w2, ca_b2,
           g_w2, g_b2, bn2_g, bn2_b, g_w3, g_b3, shared_w, shared_b,
           expert_w, expert_b):
    B, C, H, W = x.shape
    P = H * W
    hidden = fe_w.shape[1]
    h2 = g_w1.shape[1]
    E = expert_w.shape[0]
    nP = _NP
    TN = P // nP
    f32 = jnp.float32
    bf16 = jnp.bfloat16

    x3 = x.reshape(B, C, P)
    feb2 = fe_b.reshape(1, hidden)

    c = 1.0 / math.sqrt(1.0 + 1e-5)
    s1 = bn1_g * c
    w1f = g_w1 * s1[None, :]
    b1f = (g_b1 * s1 + bn1_b).reshape(1, h2)
    s2 = bn2_g * c
    w2f = g_w2 * s2[None, :]
    b2f = (g_b2 * s2 + bn2_b).reshape(1, hidden)

    red = ca_w1.shape[1]
    cw1p = jnp.pad(ca_w1, ((0, 0), (0, 128 - red)))
    cb1p = jnp.pad(ca_b1, (0, 128 - red)).reshape(1, 128)
    cw2p = jnp.pad(ca_w2, ((0, 128 - red), (0, 0)))
    cb2r = ca_b2.reshape(1, h2)
    w3p = jnp.pad(g_w3, ((0, 0), (0, 128 - E)))
    b3p = jnp.pad(g_b3, (0, 128 - E), constant_values=-1e9).reshape(1, 128)

    swb = shared_w.astype(bf16)
    ewb = expert_w.astype(bf16)
    eb3 = expert_b.reshape(E, 1, hidden)
    sb3 = shared_b.reshape(2, 1, hidden)

    cm = lambda b, j: (0, 0)
    cm3 = lambda b, j: (0, 0, 0)
    nB = B

    out = pl.pallas_call(
        functools.partial(_fused_body, nB=nB, TN=TN, inv_p=1.0 / P),
        grid=(B + 1, nP),
        in_specs=[
            pl.BlockSpec((1, C, TN),
                         lambda b, j: (jnp.minimum(b, nB - 1), 0,
                                       jnp.where(b < nB, j, nP - 1))),
            pl.BlockSpec((C, hidden), cm),
            pl.BlockSpec((1, hidden), cm),
            pl.BlockSpec((hidden, h2), cm),
            pl.BlockSpec((1, h2), cm),
            pl.BlockSpec((h2, 128), cm),
            pl.BlockSpec((1, 128), cm),
            pl.BlockSpec((128, h2), cm),
            pl.BlockSpec((1, h2), cm),
            pl.BlockSpec((h2, hidden), cm),
            pl.BlockSpec((1, hidden), cm),
            pl.BlockSpec((hidden, 128), cm),
            pl.BlockSpec((1, 128), cm),
            pl.BlockSpec((2, hidden, hidden), cm3),
            pl.BlockSpec((2, 1, hidden), cm3),
            pl.BlockSpec((E, hidden, hidden), cm3),
            pl.BlockSpec((E, 1, hidden), cm3),
        ],
        out_specs=pl.BlockSpec(
            (1, hidden, TN),
            lambda b, j: (jnp.maximum(b, 1) - 1, 0, jnp.where(b > 0, j, 0))),
        out_shape=jax.ShapeDtypeStruct((B, hidden, P), f32),
        scratch_shapes=[
            pltpu.VMEM((2, hidden, P), bf16),
            pltpu.VMEM((1, hidden), f32),
            pltpu.SMEM((2,), jnp.int32),
            pltpu.SMEM((2,), f32),
        ],
        interpret=_INTERPRET,
    )(x3, fe_w, feb2, w1f, b1f, cw1p, cb1p, cw2p, cb2r, w2f, b2f, w3p, b3p,
      swb, sb3, ewb, eb3)

    return out.reshape(B, hidden, H, W)


# speedup vs baseline: 1.8974x; 1.6257x over previous
import functools
import math

import jax
import jax.numpy as jnp
from jax.experimental import pallas as pl
from jax.experimental.pallas import tpu as pltpu

_INTERPRET = False

_NP = 7


def _gelu(x):
    return 0.5 * x * (1.0 + jax.lax.erf(x * (1.0 / math.sqrt(2.0))))


def _dotc(w, x):
    return jax.lax.dot_general(
        w, x, (((0,), (0,)), ((), ())), preferred_element_type=jnp.float32
    )


def _dot(a, b):
    return jnp.dot(a, b, preferred_element_type=jnp.float32)


def _fused_body(x_ref, few_ref, feb_ref,
                w1_ref, b1_ref, cw1_ref, cb1_ref, cw2_ref, cb2_ref,
                w2_ref, b2_ref, w3_ref, b3_ref,
                sw_ref, sb_ref, ew_ref, eb_ref,
                out_ref,
                ft_ref, acc_ref, ti_ref, tw_ref,
                *, nB, TN, inv_p):
    b = pl.program_id(0)
    j = pl.program_id(1)
    C = x_ref.shape[1]
    hidden = out_ref.shape[1]
    TH = x_ref.shape[2]
    W = x_ref.shape[3]

    @pl.when((b > 0) & (j == 0))
    def _():
        g = acc_ref[...] * inv_p
        z = _gelu(_dot(g, w1_ref[...]) + b1_ref[...])
        a = _gelu(_dot(z, cw1_ref[...]) + cb1_ref[...])
        a = _dot(a, cw2_ref[...]) + cb2_ref[...]
        z = z * jax.nn.sigmoid(2.0 * a)
        z = _gelu(_dot(z, w2_ref[...]) + b2_ref[...])
        s = _dot(z, w3_ref[...]) + b3_ref[...]
        idx = jax.lax.broadcasted_iota(jnp.int32, s.shape, 1)
        m1 = jnp.max(s)
        i1 = jnp.min(jnp.where(s >= m1, idx, 127))
        s2 = jnp.where(idx == i1, -jnp.inf, s)
        m2 = jnp.max(s2)
        i2 = jnp.min(jnp.where(s2 >= m2, idx, 127))
        d = jnp.exp((m2 - m1) * 0.5)
        ti_ref[0] = i1
        ti_ref[1] = i2
        tw_ref[0] = 1.0 / (1.0 + d)
        tw_ref[1] = d / (1.0 + d)

    @pl.when(b < nB)
    def _():
        xb = x_ref[0].reshape(C, TN)
        ft = _gelu(_dotc(few_ref[...], xb) + feb_ref[0][:, None])
        ft_ref[b % 2, :, pl.ds(j * TN, TN)] = ft.astype(jnp.bfloat16)
        s = jnp.sum(ft, axis=1)[None, :]

        @pl.when(j == 0)
        def _():
            acc_ref[...] = s

        @pl.when(j > 0)
        def _():
            acc_ref[...] = acc_ref[...] + s

    @pl.when(b > 0)
    def _():
        ftb = ft_ref[(b + 1) % 2, :, pl.ds(j * TN, TN)]
        acc = 0.5 * _gelu(_dotc(sw_ref[0], ftb) + sb_ref[0, 0][:, None])
        acc = acc + 0.5 * _gelu(_dotc(sw_ref[1], ftb) + sb_ref[1, 0][:, None])
        i0 = ti_ref[0]
        i1 = ti_ref[1]
        acc = acc + tw_ref[0] * _gelu(_dotc(ew_ref[i0], ftb)
                                      + eb_ref[i0, 0][:, None])
        acc = acc + tw_ref[1] * _gelu(_dotc(ew_ref[i1], ftb)
                                      + eb_ref[i1, 0][:, None])
        out_ref[0] = acc.reshape(hidden, TH, W)


def kernel(x, fe_w, fe_b, g_w1, g_b1, bn1_g, bn1_b, ca_w1, ca_b1, ca_w2, ca_b2,
           g_w2, g_b2, bn2_g, bn2_b, g_w3, g_b3, shared_w, shared_b,
           expert_w, expert_b):
    B, C, H, W = x.shape
    P = H * W
    hidden = fe_w.shape[1]
    h2 = g_w1.shape[1]
    E = expert_w.shape[0]
    nP = _NP
    TN = P // nP
    TH = H // nP
    f32 = jnp.float32
    bf16 = jnp.bfloat16

    feb2 = fe_b.reshape(1, hidden)

    c = 1.0 / math.sqrt(1.0 + 1e-5)
    s1 = bn1_g * c
    w1f = g_w1 * s1[None, :]
    b1f = (g_b1 * s1 + bn1_b).reshape(1, h2)
    s2 = bn2_g * c
    w2f = g_w2 * s2[None, :]
    b2f = (g_b2 * s2 + bn2_b).reshape(1, hidden)

    red = ca_w1.shape[1]
    cw1p = jnp.pad(ca_w1, ((0, 0), (0, 128 - red)))
    cb1p = jnp.pad(ca_b1, (0, 128 - red)).reshape(1, 128)
    cw2p = jnp.pad(ca_w2, ((0, 128 - red), (0, 0)))
    cb2r = ca_b2.reshape(1, h2)
    w3p = jnp.pad(g_w3, ((0, 0), (0, 128 - E)))
    b3p = jnp.pad(g_b3, (0, 128 - E), constant_values=-1e9).reshape(1, 128)

    swb = shared_w.astype(bf16)
    ewb = expert_w.astype(bf16)
    eb3 = expert_b.reshape(E, 1, hidden)
    sb3 = shared_b.reshape(2, 1, hidden)

    cm = lambda b, j: (0, 0)
    cm3 = lambda b, j: (0, 0, 0)
    nB = B

    out = pl.pallas_call(
        functools.partial(_fused_body, nB=nB, TN=TN, inv_p=1.0 / P),
        grid=(B + 1, nP),
        in_specs=[
            pl.BlockSpec((1, C, TH, W),
                         lambda b, j: (jnp.minimum(b, nB - 1), 0,
                                       jnp.where(b < nB, j, nP - 1), 0)),
            pl.BlockSpec((C, hidden), cm),
            pl.BlockSpec((1, hidden), cm),
            pl.BlockSpec((hidden, h2), cm),
            pl.BlockSpec((1, h2), cm),
            pl.BlockSpec((h2, 128), cm),
            pl.BlockSpec((1, 128), cm),
            pl.BlockSpec((128, h2), cm),
            pl.BlockSpec((1, h2), cm),
            pl.BlockSpec((h2, hidden), cm),
            pl.BlockSpec((1, hidden), cm),
            pl.BlockSpec((hidden, 128), cm),
            pl.BlockSpec((1, 128), cm),
            pl.BlockSpec((2, hidden, hidden), cm3),
            pl.BlockSpec((2, 1, hidden), cm3),
            pl.BlockSpec((E, hidden, hidden), cm3),
            pl.BlockSpec((E, 1, hidden), cm3),
        ],
        out_specs=pl.BlockSpec(
            (1, hidden, TH, W),
            lambda b, j: (jnp.maximum(b, 1) - 1, 0, jnp.where(b > 0, j, 0), 0)),
        out_shape=jax.ShapeDtypeStruct((B, hidden, H, W), f32),
        scratch_shapes=[
            pltpu.VMEM((2, hidden, P), bf16),
            pltpu.VMEM((1, hidden), f32),
            pltpu.SMEM((2,), jnp.int32),
            pltpu.SMEM((2,), f32),
        ],
        interpret=_INTERPRET,
    )(x, fe_w, feb2, w1f, b1f, cw1p, cb1p, cw2p, cb2r, w2f, b2f, w3p, b3p,
      swb, sb3, ewb, eb3)

    return out
